# baseline (speedup 1.0000x reference)
import jax
import jax.numpy as jnp
from jax import lax
from jax.experimental import pallas as pl
from jax.experimental.pallas import tpu as pltpu

_IPS = 4


def _make_fused_kernel(N, Cin, Cout, KH, KW, pad, Ho, Wo, eps, ips):
    HoWo = Ho * Wo
    K = KH * KW * Cin
    inv_count = 1.0 / float(N * HoWo)
    nconv = N // ips

    def body(x_ref, w_ref, g_ref, b_ref, o_ref, y_ref, sum_ref, ssq_ref):
        i = pl.program_id(0)

        @pl.when(i == 0)
        def _init():
            sum_ref[...] = jnp.zeros_like(sum_ref)
            ssq_ref[...] = jnp.zeros_like(ssq_ref)

        @pl.when(i < nconv)
        def _conv():
            for j in range(ips):
                patches = jnp.concatenate(
                    [x_ref[j][kh:kh + Ho, :, :] for kh in range(KH)],
                    axis=2).reshape(HoWo, K)
                y = lax.dot_general(
                    w_ref[...], patches,
                    dimension_numbers=(((1,), (1,)), ((), ())),
                    preferred_element_type=jnp.float32)
                y_ref[i * ips + j] = y.astype(jnp.bfloat16)
                sum_ref[...] += jnp.sum(y, axis=1, keepdims=True)
                ssq_ref[...] += jnp.sum(y * y, axis=1, keepdims=True)

        @pl.when(i >= nconv)
        def _apply():
            mean = sum_ref[...] * inv_count
            var = jnp.maximum(ssq_ref[...] * inv_count - mean * mean, 0.0)
            scale = g_ref[...] * lax.rsqrt(var + eps)
            shift = b_ref[...] - mean * scale
            base = (i - nconv) * ips
            for j in range(ips):
                o_ref[j] = y_ref[base + j].astype(jnp.float32) * scale + shift

    return body


def kernel(x_nchw, w_oihw, gamma, beta):
    stride, pad, eps = 1, 1, 1e-5
    N, Cin, H, W = x_nchw.shape
    Cout, _, KH, KW = w_oihw.shape
    Ho = (H + 2 * pad - KH) // stride + 1
    Wo = (W + 2 * pad - KW) // stride + 1
    Hp = H + 2 * pad
    HoWo = Ho * Wo
    K = KH * KW * Cin
    ips = _IPS if N % _IPS == 0 else 1
    nconv = N // ips

    x_nhwc = jnp.transpose(x_nchw, (0, 2, 3, 1)).astype(jnp.bfloat16)
    xpad = jnp.pad(x_nhwc, ((0, 0), (pad, pad), (pad, pad), (0, 0)))
    xpw = jnp.concatenate(
        [xpad[:, :, kw:kw + Wo, :] for kw in range(KW)], axis=3)
    w_ck = jnp.transpose(w_oihw, (0, 2, 3, 1)).reshape(Cout, K).astype(
        jnp.bfloat16)
    g_col = gamma.astype(jnp.float32).reshape(Cout, 1)
    b_col = beta.astype(jnp.float32).reshape(Cout, 1)

    body = _make_fused_kernel(N, Cin, Cout, KH, KW, pad, Ho, Wo, eps, ips)

    out = pl.pallas_call(
        body,
        out_shape=jax.ShapeDtypeStruct((N, Cout, HoWo), jnp.float32),
        grid=(2 * nconv,),
        in_specs=[
            pl.BlockSpec((ips, Hp, Wo, KW * Cin),
                         lambda i: (jnp.minimum(i, nconv - 1), 0, 0, 0)),
            pl.BlockSpec((Cout, K), lambda i: (0, 0)),
            pl.BlockSpec((Cout, 1), lambda i: (0, 0)),
            pl.BlockSpec((Cout, 1), lambda i: (0, 0)),
        ],
        out_specs=pl.BlockSpec((ips, Cout, HoWo),
                               lambda i: (jnp.maximum(i - nconv, 0), 0, 0)),
        scratch_shapes=[
            pltpu.VMEM((N, Cout, HoWo), jnp.bfloat16),
            pltpu.VMEM((Cout, 1), jnp.float32),
            pltpu.VMEM((Cout, 1), jnp.float32),
        ],
        compiler_params=pltpu.CompilerParams(
            dimension_semantics=("arbitrary",),
            vmem_limit_bytes=56 * 1024 * 1024,
        ),
        name="conv_bn_fused",
    )(xpw, w_ck, g_col, b_col)

    return out.reshape(N, Cout, Ho, Wo)


# speedup vs baseline: 1.6041x; 1.6041x over previous
import jax
import jax.numpy as jnp
from jax import lax
from jax.experimental import pallas as pl
from jax.experimental.pallas import tpu as pltpu

_IPS = 4


def _make_fused_kernel(N, H, W, Cin, Cout, KH, KW, pad, Ho, Wo, eps, ips):
    Hp = H + 2 * pad
    HoWo = Ho * Wo
    K = KH * KW * Cin
    inv_count = 1.0 / float(N * HoWo)
    nconv = N // ips

    def body(x_ref, w_ref, g_ref, b_ref, o_ref,
             pw_ref, y_ref, sum_ref, ssq_ref):
        i = pl.program_id(0)

        @pl.when(i == 0)
        def _init():
            sum_ref[...] = jnp.zeros_like(sum_ref)
            ssq_ref[...] = jnp.zeros_like(ssq_ref)
            pw_ref[...] = jnp.zeros_like(pw_ref)

        @pl.when(i < nconv)
        def _conv():
            for j in range(ips):
                for kw in range(KW):
                    lo = max(pad - kw, 0)
                    hi = min(W + pad - kw, Wo)
                    pw_ref[pad:pad + H, lo:hi, kw * Cin:(kw + 1) * Cin] = (
                        x_ref[j][:, lo + kw - pad:hi + kw - pad, :])
                patches = jnp.concatenate(
                    [pw_ref[kh:kh + Ho, :, :] for kh in range(KH)],
                    axis=2).reshape(HoWo, K)
                y = lax.dot_general(
                    w_ref[...], patches,
                    dimension_numbers=(((1,), (1,)), ((), ())),
                    preferred_element_type=jnp.float32)
                y_ref[i * ips + j] = y.astype(jnp.bfloat16)
                sum_ref[...] += jnp.sum(y, axis=1, keepdims=True)
                ssq_ref[...] += jnp.sum(y * y, axis=1, keepdims=True)

        @pl.when(i >= nconv)
        def _apply():
            mean = sum_ref[...] * inv_count
            var = jnp.maximum(ssq_ref[...] * inv_count - mean * mean, 0.0)
            scale = g_ref[...] * lax.rsqrt(var + eps)
            shift = b_ref[...] - mean * scale
            base = (i - nconv) * ips
            for j in range(ips):
                o_ref[j] = y_ref[base + j].astype(jnp.float32) * scale + shift

    return body


def kernel(x_nchw, w_oihw, gamma, beta):
    stride, pad, eps = 1, 1, 1e-5
    N, Cin, H, W = x_nchw.shape
    Cout, _, KH, KW = w_oihw.shape
    Ho = (H + 2 * pad - KH) // stride + 1
    Wo = (W + 2 * pad - KW) // stride + 1
    HoWo = Ho * Wo
    K = KH * KW * Cin
    ips = _IPS if N % _IPS == 0 else 1
    nconv = N // ips

    x_nhwc = jnp.transpose(x_nchw, (0, 2, 3, 1)).astype(jnp.bfloat16)
    w_ck = jnp.transpose(w_oihw, (0, 2, 3, 1)).reshape(Cout, K).astype(
        jnp.bfloat16)
    g_col = gamma.astype(jnp.float32).reshape(Cout, 1)
    b_col = beta.astype(jnp.float32).reshape(Cout, 1)

    body = _make_fused_kernel(N, H, W, Cin, Cout, KH, KW, pad, Ho, Wo, eps,
                              ips)

    out = pl.pallas_call(
        body,
        out_shape=jax.ShapeDtypeStruct((N, Cout, HoWo), jnp.float32),
        grid=(2 * nconv,),
        in_specs=[
            pl.BlockSpec((ips, H, W, Cin),
                         lambda i: (jnp.minimum(i, nconv - 1), 0, 0, 0)),
            pl.BlockSpec((Cout, K), lambda i: (0, 0)),
            pl.BlockSpec((Cout, 1), lambda i: (0, 0)),
            pl.BlockSpec((Cout, 1), lambda i: (0, 0)),
        ],
        out_specs=pl.BlockSpec((ips, Cout, HoWo),
                               lambda i: (jnp.maximum(i - nconv, 0), 0, 0)),
        scratch_shapes=[
            pltpu.VMEM((H + 2 * pad, Wo, KW * Cin),
                       jnp.bfloat16),
            pltpu.VMEM((N, Cout, HoWo), jnp.bfloat16),
            pltpu.VMEM((Cout, 1), jnp.float32),
            pltpu.VMEM((Cout, 1), jnp.float32),
        ],
        compiler_params=pltpu.CompilerParams(
            dimension_semantics=("arbitrary",),
            vmem_limit_bytes=56 * 1024 * 1024,
        ),
        name="conv_bn_fused",
    )(x_nhwc, w_ck, g_col, b_col)

    return out.reshape(N, Cout, Ho, Wo)
